# baseline (speedup 1.0000x reference)
import jax
import jax.numpy as jnp
from jax.experimental import pallas as pl
from jax.experimental.pallas import tpu as pltpu

_N = 1000
_B = 256
_CHUNKS_COMP = 2
_CHUNKS_MEM = 6


def _enqueue_kernel(comp_ref, mem_ref, out_ref, sems):
    copies = []
    rc = _B // _CHUNKS_COMP
    for c in range(_CHUNKS_COMP):
        copies.append(pltpu.make_async_copy(
            comp_ref.at[pl.ds(c * rc, rc)],
            out_ref.at[pl.ds(c * rc, rc)],
            sems.at[c]))
    rm = (_N - _B) // _CHUNKS_MEM
    for c in range(_CHUNKS_MEM):
        copies.append(pltpu.make_async_copy(
            mem_ref.at[pl.ds(_B + c * rm, rm)],
            out_ref.at[pl.ds(_B + c * rm, rm)],
            sems.at[_CHUNKS_COMP + c]))
    for cp in copies:
        cp.start()
    for cp in copies:
        cp.wait()


def kernel(memory_bank, components):
    comps = jax.lax.stop_gradient(components)
    return pl.pallas_call(
        _enqueue_kernel,
        in_specs=[
            pl.BlockSpec(memory_space=pltpu.MemorySpace.HBM),
            pl.BlockSpec(memory_space=pltpu.MemorySpace.HBM),
        ],
        out_specs=pl.BlockSpec(memory_space=pltpu.MemorySpace.HBM),
        out_shape=jax.ShapeDtypeStruct((_N, 256, 256), memory_bank.dtype),
        scratch_shapes=[pltpu.SemaphoreType.DMA((_CHUNKS_COMP + _CHUNKS_MEM,))],
    )(comps, memory_bank)


# speedup vs baseline: 44.2620x; 44.2620x over previous
import jax
import jax.numpy as jnp
from jax.experimental import pallas as pl
from jax.experimental.pallas import tpu as pltpu

_N = 1000
_B = 256
_R = 8
_NB = _N // _R
_NB_COMP = _B // _R


def _enqueue_kernel(comp_ref, mem_ref, out_ref):
    i = pl.program_id(0)

    @pl.when(i < _NB_COMP)
    def _():
        out_ref[...] = comp_ref[...]

    @pl.when(i >= _NB_COMP)
    def _():
        out_ref[...] = mem_ref[...]


def kernel(memory_bank, components):
    comps = jax.lax.stop_gradient(components)
    return pl.pallas_call(
        _enqueue_kernel,
        grid=(_NB,),
        in_specs=[
            pl.BlockSpec((_R, 256, 256), lambda i: (jnp.minimum(i, _NB_COMP - 1), 0, 0)),
            pl.BlockSpec((_R, 256, 256), lambda i: (jnp.maximum(i, _NB_COMP), 0, 0)),
        ],
        out_specs=pl.BlockSpec((_R, 256, 256), lambda i: (i, 0, 0)),
        out_shape=jax.ShapeDtypeStruct((_N, 256, 256), memory_bank.dtype),
        compiler_params=pltpu.CompilerParams(
            dimension_semantics=("parallel",),
        ),
    )(comps, memory_bank)
